# Initial kernel scaffold; baseline (speedup 1.0000x reference)
import jax
import jax.numpy as jnp
from jax.experimental import pallas as pl


def kernel(updates, mask):
    raise NotImplementedError("write your pallas kernel here")


# speedup vs baseline: 16.0481x; 16.0481x over previous
import functools

import jax
import jax.numpy as jnp
from jax import lax
from jax.experimental import pallas as pl
from jax.experimental.pallas import tpu as pltpu
from jax.experimental.pallas import tpu_sc as plsc

B, H, W, C = 4, 112, 112, 192
HW = H * W
P = (2 * H) * (2 * W)
CB = 16
NCB = C // CB
NC, NS, L = 2, 16, 16
ROWS = HW // NS
HALF = ROWS // 2
CH = HALF * CB // 128
ACC = P * CB
POS_T = P // NS
RB_POS = POS_T // 4
RB_N = RB_POS * CB
ZN = RB_N // 2

_mesh = plsc.VectorSubcoreMesh(core_axis_name="c", subcore_axis_name="s")


@functools.partial(
    pl.kernel,
    out_type=jax.ShapeDtypeStruct((B, P, C), jnp.float32),
    mesh=_mesh,
    compiler_params=pltpu.CompilerParams(use_tc_tiling_on_sc=False),
    scratch_types=[
        pltpu.VMEM_SHARED((ACC,), jnp.float32),
        pltpu.VMEM((HALF, CB), jnp.int32),
        pltpu.VMEM((HALF, CB), jnp.float32),
        pltpu.VMEM((CH, 128), jnp.int32),
        pltpu.VMEM((CH, 128), jnp.float32),
        pltpu.VMEM((RB_N,), jnp.float32),
        pltpu.VMEM((RB_POS, CB), jnp.float32),
        pltpu.VMEM((ZN,), jnp.float32),
    ],
)
def _unpool(upd_hbm, msk_hbm, out_hbm, acc, msk_w, upd_w, idx2d, val2d,
            rb1d, rb2d, zeros):
    core = lax.axis_index("c")
    sub = lax.axis_index("s")
    row0 = sub * ROWS
    pos0 = sub * POS_T
    iota = lax.broadcasted_iota(jnp.int32, (L,), 0)
    z16 = jnp.zeros((L,), jnp.float32)
    third = jnp.float32(1.0 / 3.0)

    @pl.loop(0, ZN // L)
    def _(g):
        zeros[pl.ds(g * L, L)] = z16

    @pl.loop(0, POS_T * CB // ZN)
    def _(q):
        pltpu.sync_copy(zeros, acc.at[pl.ds(pos0 * CB + q * ZN, ZN)])

    plsc.subcore_barrier()

    @pl.loop(0, B)
    def _(b):
        @pl.loop(0, NCB // NC)
        def _(cbs):
            c0 = cbs * (CB * NC) + core * CB
            for h in range(2):
                r0 = row0 + h * HALF
                pltpu.sync_copy(msk_hbm.at[b, pl.ds(r0, HALF), pl.ds(c0, CB)],
                                msk_w)
                pltpu.sync_copy(upd_hbm.at[b, pl.ds(r0, HALF), pl.ds(c0, CB)],
                                upd_w)

                @pl.loop(0, CH)
                def _(j):
                    for jj in range(8):
                        row = j * 8 + jj
                        vm = msk_w[row, pl.ds(0, L)]
                        u = lax.shift_right_logical(vm, 6)
                        p = (u.astype(jnp.float32) * third).astype(jnp.int32)
                        idx2d[j, pl.ds(jj * L, L)] = p * CB + iota
                        val2d[j, pl.ds(jj * L, L)] = upd_w[row, pl.ds(0, L)]

                @pl.loop(0, CH)
                def _(j):
                    pltpu.sync_copy(val2d.at[j], acc.at[idx2d.at[j]],
                                    add=True)

            plsc.subcore_barrier()

            @pl.loop(0, POS_T // RB_POS)
            def _(q):
                off = (pos0 + q * RB_POS) * CB
                pltpu.sync_copy(acc.at[pl.ds(off, RB_N)], rb1d)

                @pl.loop(0, RB_POS)
                def _(t):
                    rb2d[t, pl.ds(0, L)] = rb1d[pl.ds(t * L, L)]

                pltpu.sync_copy(rb2d,
                                out_hbm.at[b, pl.ds(pos0 + q * RB_POS, RB_POS),
                                           pl.ds(c0, CB)])
                pltpu.sync_copy(zeros, acc.at[pl.ds(off, ZN)])
                pltpu.sync_copy(zeros, acc.at[pl.ds(off + ZN, ZN)])

            plsc.subcore_barrier()


def kernel(updates, mask):
    upd3 = updates.reshape(B, HW, C)
    msk3 = mask.astype(jnp.int32).reshape(B, HW, C)
    out = _unpool(upd3, msk3)
    return out.reshape(B, 2 * H, 2 * W, C)
